# baseline (speedup 1.0000x reference)
import functools

import jax
import jax.numpy as jnp
from jax import lax
from jax.experimental import pallas as pl
from jax.experimental.pallas import tpu as pltpu
from jax.experimental.pallas import tpu_sc as plsc

EMBED_DIM = 64
NUM_WORKERS = 32
CHUNK_B = 16


def _gather_body(
    idx_hbm, table_hbm, out_hbm, idx_v, rows0, rows1, g0, g1, o0, o1,
    *, batch, hist
):
    b_per_w = batch // NUM_WORKERS
    n_per_w = b_per_w * hist
    chunk = CHUNK_B * hist
    n_chunks = b_per_w // CHUNK_B
    n_pairs = n_chunks // 2
    wid = lax.axis_index("s") * 2 + lax.axis_index("c")
    base_b = wid * b_per_w
    pltpu.sync_copy(idx_hbm.at[pl.ds(base_b * hist, n_per_w)], idx_v)

    def gather(j, buf, sem):
        pltpu.async_copy(table_hbm.at[idx_v.at[pl.ds(j * chunk, chunk)]], buf, sem)

    def wait_gather(buf, sem):
        pltpu.make_async_copy(
            table_hbm.at[idx_v.at[pl.ds(0, chunk)]], buf, sem
        ).wait()

    def put(j, buf, sem):
        b0 = base_b + j * CHUNK_B

        def one(i, carry):
            pltpu.async_copy(
                buf.at[pl.ds(i * hist, hist)], out_hbm.at[b0 + i], sem
            )
            return carry

        lax.fori_loop(0, CHUNK_B, one, 0)

    def wait_put(buf, sem):
        def one(i, carry):
            pltpu.make_async_copy(
                buf.at[pl.ds(0, hist)], out_hbm.at[base_b], sem
            ).wait()
            return carry

        lax.fori_loop(0, CHUNK_B, one, 0)

    gather(0, rows0, g0)
    gather(1, rows1, g1)

    def body(i, carry):
        j = i * 2
        wait_gather(rows0, g0)
        put(j, rows0, o0)
        wait_gather(rows1, g1)
        put(j + 1, rows1, o1)
        wait_put(rows0, o0)
        gather(j + 2, rows0, g0)
        wait_put(rows1, o1)
        gather(j + 3, rows1, g1)
        return carry

    lax.fori_loop(0, n_pairs - 1, body, 0)

    j = n_chunks - 2
    wait_gather(rows0, g0)
    put(j, rows0, o0)
    wait_gather(rows1, g1)
    put(j + 1, rows1, o1)
    wait_put(rows0, o0)
    wait_put(rows1, o1)


def kernel(x, weight):
    batch, hist = x.shape
    idx = x.reshape(batch * hist).astype(jnp.int32)

    mesh = plsc.VectorSubcoreMesh(core_axis_name="c", subcore_axis_name="s")
    gather = functools.partial(
        pl.kernel,
        mesh=mesh,
        out_type=jax.ShapeDtypeStruct((batch, hist, EMBED_DIM), jnp.float32),
        scratch_types=[
            pltpu.VMEM((batch // NUM_WORKERS * hist,), jnp.int32),
            pltpu.VMEM((CHUNK_B * hist, EMBED_DIM), jnp.float32),
            pltpu.VMEM((CHUNK_B * hist, EMBED_DIM), jnp.float32),
            pltpu.SemaphoreType.DMA,
            pltpu.SemaphoreType.DMA,
            pltpu.SemaphoreType.DMA,
            pltpu.SemaphoreType.DMA,
        ],
        compiler_params=pltpu.CompilerParams(use_tc_tiling_on_sc=False),
    )(functools.partial(_gather_body, batch=batch, hist=hist))

    return gather(idx, weight)


# speedup vs baseline: 1.2648x; 1.2648x over previous
import functools

import jax
import jax.numpy as jnp
from jax import lax
from jax.experimental import pallas as pl
from jax.experimental.pallas import tpu as pltpu
from jax.experimental.pallas import tpu_sc as plsc

EMBED_DIM = 64
NUM_WORKERS = 32
BL = 128


def _gather_body(idx_hbm, table_hbm, out_hbm, idx_v, rows0, rows1, tb0, tb1,
                 g0, g1, o0, o1, *, batch, hist):
    n_bb = batch // BL
    bb_per_w = n_bb // NUM_WORKERS
    n_t = bb_per_w * hist
    wid = lax.axis_index("s") * 2 + lax.axis_index("c")

    pltpu.sync_copy(idx_hbm.at[pl.ds(wid * n_t, n_t)], idx_v)

    lanes = lax.broadcasted_iota(jnp.int32, (16,), 0)
    colbases = [(lanes + k) % 16 for k in range(16)]

    def gather(t, rows, sem):
        pltpu.async_copy(table_hbm.at[idx_v.at[t]], rows, sem)

    def wait_gather(rows, sem):
        pltpu.make_async_copy(table_hbm.at[idx_v.at[0]], rows, sem).wait()

    def transpose(rows, tb):
        def tp(p, c1):
            rowv = p * 16 + lanes

            def tq(q, c2):
                d0 = q * 16
                for k in range(16):
                    col = d0 + colbases[k]
                    v = plsc.load_gather(rows, [rowv, col])
                    plsc.store_scatter(tb, [col, rowv], v)
                return c2

            lax.fori_loop(0, EMBED_DIM // 16, tq, c1)
            return c1

        lax.fori_loop(0, BL // 16, tp, 0)

    def put(t, tb, sem):
        bb_local = t // hist
        h = t - bb_local * hist
        bb = wid * bb_per_w + bb_local

        def pp(db, c):
            pltpu.async_copy(tb.at[pl.ds(db * 8, 8)], out_hbm.at[h, db, bb], sem)
            return c

        lax.fori_loop(0, EMBED_DIM // 8, pp, 0)

    def wait_put(tb, sem):
        def pw(db, c):
            pltpu.make_async_copy(
                tb.at[pl.ds(0, 8)], out_hbm.at[0, 0, 0], sem
            ).wait()
            return c

        lax.fori_loop(0, EMBED_DIM // 8, pw, 0)

    gather(0, rows0, g0)
    gather(1, rows1, g1)

    wait_gather(rows0, g0)
    transpose(rows0, tb0)
    gather(2, rows0, g0)
    put(0, tb0, o0)
    wait_gather(rows1, g1)
    transpose(rows1, tb1)
    gather(3, rows1, g1)
    put(1, tb1, o1)

    def body(tt, carry):
        t0 = tt * 2
        wait_gather(rows0, g0)
        wait_put(tb0, o0)
        transpose(rows0, tb0)

        @pl.when(t0 + 2 < n_t)
        def _():
            gather(t0 + 2, rows0, g0)

        put(t0, tb0, o0)
        wait_gather(rows1, g1)
        wait_put(tb1, o1)
        transpose(rows1, tb1)

        @pl.when(t0 + 3 < n_t)
        def _():
            gather(t0 + 3, rows1, g1)

        put(t0 + 1, tb1, o1)
        return carry

    lax.fori_loop(1, n_t // 2, body, 0)

    wait_put(tb0, o0)
    wait_put(tb1, o1)


def kernel(x, weight):
    batch, hist = x.shape
    n_bb = batch // BL
    idxb = (
        x.astype(jnp.int32).T.reshape(hist, n_bb, BL)
        .transpose(1, 0, 2)
        .reshape(n_bb * hist, BL)
    )

    mesh = plsc.VectorSubcoreMesh(core_axis_name="c", subcore_axis_name="s")
    gather = functools.partial(
        pl.kernel,
        mesh=mesh,
        out_type=jax.ShapeDtypeStruct(
            (hist, EMBED_DIM // 8, n_bb, 8, BL), jnp.float32
        ),
        scratch_types=[
            pltpu.VMEM((n_bb * hist // NUM_WORKERS, BL), jnp.int32),
            pltpu.VMEM((BL, EMBED_DIM), jnp.float32),
            pltpu.VMEM((BL, EMBED_DIM), jnp.float32),
            pltpu.VMEM((EMBED_DIM, BL), jnp.float32),
            pltpu.VMEM((EMBED_DIM, BL), jnp.float32),
            pltpu.SemaphoreType.DMA,
            pltpu.SemaphoreType.DMA,
            pltpu.SemaphoreType.DMA,
            pltpu.SemaphoreType.DMA,
        ],
        compiler_params=pltpu.CompilerParams(
            use_tc_tiling_on_sc=False, needs_layout_passes=False
        ),
    )(functools.partial(_gather_body, batch=batch, hist=hist))

    out5 = gather(idxb, weight)
    return out5.transpose(2, 4, 0, 1, 3).reshape(batch, hist, EMBED_DIM)


# speedup vs baseline: 1.6760x; 1.3251x over previous
import functools

import jax
import jax.numpy as jnp
from jax import lax
from jax.experimental import layout as jax_layout
from jax.experimental import pallas as pl
from jax.experimental.pallas import tpu as pltpu
from jax.experimental.pallas import tpu_sc as plsc

EMBED_DIM = 64
NUM_WORKERS = 32
BL = 128


def _gather_body(idx_hbm, table_hbm, out_hbm, idx_v, rows0, rows1, tb0, tb1,
                 g0, g1, o0, o1, *, batch, hist):
    n_bb = batch // BL
    bb_per_w = n_bb // NUM_WORKERS
    n_t = bb_per_w * hist
    wid = lax.axis_index("s") * 2 + lax.axis_index("c")

    pltpu.sync_copy(idx_hbm.at[pl.ds(wid * n_t, n_t)], idx_v)

    lanes = lax.broadcasted_iota(jnp.int32, (16,), 0)
    colbases = [(lanes + k) % 16 for k in range(16)]

    def gather(t, rows, sem):
        pltpu.async_copy(table_hbm.at[idx_v.at[t]], rows, sem)

    def wait_gather(rows, sem):
        pltpu.make_async_copy(table_hbm.at[idx_v.at[0]], rows, sem).wait()

    def transpose(rows, tb):
        def tp(p, c1):
            rowv = p * 16 + lanes

            def tq(q, c2):
                d0 = q * 16
                for k in range(16):
                    col = d0 + colbases[k]
                    v = plsc.load_gather(rows, [rowv, col])
                    plsc.store_scatter(tb, [col, rowv], v)
                return c2

            lax.fori_loop(0, EMBED_DIM // 16, tq, c1)
            return c1

        lax.fori_loop(0, BL // 16, tp, 0)

    def put(t, tb, sem):
        bb_local = t // hist
        h = t - bb_local * hist
        bb = wid * bb_per_w + bb_local

        def pp(db, c):
            pltpu.async_copy(tb.at[pl.ds(db * 8, 8)], out_hbm.at[h, db, bb], sem)
            return c

        lax.fori_loop(0, EMBED_DIM // 8, pp, 0)

    def wait_put(tb, sem):
        def pw(db, c):
            pltpu.make_async_copy(
                tb.at[pl.ds(0, 8)], out_hbm.at[0, 0, 0], sem
            ).wait()
            return c

        lax.fori_loop(0, EMBED_DIM // 8, pw, 0)

    gather(0, rows0, g0)
    gather(1, rows1, g1)

    wait_gather(rows0, g0)
    transpose(rows0, tb0)
    gather(2, rows0, g0)
    put(0, tb0, o0)
    wait_gather(rows1, g1)
    transpose(rows1, tb1)
    gather(3, rows1, g1)
    put(1, tb1, o1)

    def body(tt, carry):
        t0 = tt * 2
        wait_gather(rows0, g0)
        wait_put(tb0, o0)
        transpose(rows0, tb0)

        @pl.when(t0 + 2 < n_t)
        def _():
            gather(t0 + 2, rows0, g0)

        put(t0, tb0, o0)
        wait_gather(rows1, g1)
        wait_put(tb1, o1)
        transpose(rows1, tb1)

        @pl.when(t0 + 3 < n_t)
        def _():
            gather(t0 + 3, rows1, g1)

        put(t0 + 1, tb1, o1)
        return carry

    lax.fori_loop(1, n_t // 2, body, 0)

    wait_put(tb0, o0)
    wait_put(tb1, o1)


def kernel(x, weight):
    batch, hist = x.shape
    n_bb = batch // BL
    weight = jax_layout.with_layout_constraint(
        weight,
        jax_layout.Layout(major_to_minor=(0, 1), tiling=((8,),)),
    )
    idxb = (
        x.astype(jnp.int32).T.reshape(hist, n_bb, BL)
        .transpose(1, 0, 2)
        .reshape(n_bb * hist, BL)
    )

    mesh = plsc.VectorSubcoreMesh(core_axis_name="c", subcore_axis_name="s")
    gather = functools.partial(
        pl.kernel,
        mesh=mesh,
        out_type=jax.ShapeDtypeStruct(
            (hist, EMBED_DIM // 8, n_bb, 8, BL), jnp.float32
        ),
        scratch_types=[
            pltpu.VMEM((n_bb * hist // NUM_WORKERS, BL), jnp.int32),
            pltpu.VMEM((BL, EMBED_DIM), jnp.float32),
            pltpu.VMEM((BL, EMBED_DIM), jnp.float32),
            pltpu.VMEM((EMBED_DIM, BL), jnp.float32),
            pltpu.VMEM((EMBED_DIM, BL), jnp.float32),
            pltpu.SemaphoreType.DMA,
            pltpu.SemaphoreType.DMA,
            pltpu.SemaphoreType.DMA,
            pltpu.SemaphoreType.DMA,
        ],
        compiler_params=pltpu.CompilerParams(
            use_tc_tiling_on_sc=False, needs_layout_passes=False
        ),
    )(functools.partial(_gather_body, batch=batch, hist=hist))

    out5 = gather(idxb, weight)
    return out5.transpose(2, 4, 0, 1, 3).reshape(batch, hist, EMBED_DIM)
